# baseline (speedup 1.0000x reference)
import functools

import jax
import jax.numpy as jnp
from jax.experimental import pallas as pl
from jax.experimental.pallas import tpu as pltpu
from jax.experimental.pallas import tpu_sc as plsc

B, T, D = 4, 2048, 1024
E = 8
EMB = 64
RH = 128
FF = 4096
NS, NV = 5, 4
L = 16

BF = 512
NJ = FF // BF

_O_SE = 0
_O_VE = _O_SE + NS * EMB
_O_RW1 = _O_VE + NV * EMB
_O_RB1 = _O_RW1 + 2 * EMB * RH
_O_RW2 = _O_RB1 + RH
_O_RB2 = _O_RW2 + RH * L
_FLAT_N = _O_RB2 + L


def _sc_router_body(ids_hbm, flat_hbm, probs_hbm, sel_hbm, loss_hbm,
                    ids_v, flat_v, probs_v, sel_v, loss_v):
    cid = jax.lax.axis_index("c")
    scid = jax.lax.axis_index("s")
    wid = scid * 2 + cid

    @pl.when(wid == 0)
    def _():
        pltpu.sync_copy(ids_hbm, ids_v)
        pltpu.sync_copy(flat_hbm, flat_v)
        ids_row = ids_v[...]
        psum = jnp.zeros((L,), jnp.float32)
        cnts = [jnp.float32(0.0)] * E
        for b in range(B):
            s_id = ids_row[b]
            v_id = ids_row[B + b]
            for c in range(EMB // L):
                acc = jnp.zeros((L,), jnp.float32)
                for s in range(NS):
                    row = flat_v[pl.ds(_O_SE + s * EMB + c * L, L)]
                    acc = acc + (s_id == s).astype(jnp.float32) * row
                probs_v[B + c, :] = acc
            for c in range(EMB // L):
                acc = jnp.zeros((L,), jnp.float32)
                for v in range(NV):
                    row = flat_v[pl.ds(_O_VE + v * EMB + c * L, L)]
                    acc = acc + (v_id == v).astype(jnp.float32) * row
                probs_v[B + EMB // L + c, :] = acc

            def l1_body(kc, accs):
                zc = probs_v[B + kc, :]
                for l in range(L):
                    zk = zc[l]
                    ko = kc * L + l
                    accs = tuple(
                        accs[c] + zk * flat_v[pl.ds(_O_RW1 + ko * RH + c * L, L)]
                        for c in range(RH // L))
                return accs

            h = jax.lax.fori_loop(
                0, 2 * EMB // L, l1_body,
                tuple(flat_v[pl.ds(_O_RB1 + c * L, L)] for c in range(RH // L)))
            for c in range(RH // L):
                probs_v[B + c, :] = jnp.maximum(h[c], 0.0)

            def l2_body(kc, acc):
                hc = probs_v[B + kc, :]
                for l in range(L):
                    ko = kc * L + l
                    acc = acc + hc[l] * flat_v[pl.ds(_O_RW2 + ko * L, L)]
                return acc

            logits = jax.lax.fori_loop(0, RH // L, l2_body,
                                       flat_v[pl.ds(_O_RB2, L)])
            m = logits[0]
            for l in range(1, L):
                m = jnp.maximum(m, logits[l])
            ex = jnp.exp(logits - m)
            ssum = ex[0]
            for l in range(1, L):
                ssum = ssum + ex[l]
            pr = ex / jax.lax.broadcast(ssum, (L,))
            probs_v[b, :] = pr
            selb = jnp.int32(E - 1)
            bestv = pr[E - 1]
            for e in range(E - 2, -1, -1):
                pe = pr[e]
                take = pe >= bestv
                selb = jnp.where(take, jnp.int32(e), selb)
                bestv = jnp.where(take, pe, bestv)
            sel_v[b, :] = jax.lax.broadcast(selb, (L,))
            for e in range(E):
                cnts[e] = cnts[e] + (selb == e).astype(jnp.float32)
            psum = psum + pr
        loss = jnp.float32(0.0)
        for e in range(E):
            loss = loss + cnts[e] * psum[e]
        loss = loss * (E / (B * B))
        loss_v[...] = jax.lax.broadcast(loss, (L,))
        pltpu.sync_copy(probs_v, probs_hbm)
        pltpu.sync_copy(sel_v, sel_hbm)
        pltpu.sync_copy(loss_v, loss_hbm)


def _sc_router(stage_ids, view_ids, stage_emb, view_emb, rw1, rb1, rw2, rb2):
    ids = jnp.zeros((L,), jnp.int32)
    ids = ids.at[:B].set(stage_ids.astype(jnp.int32))
    ids = ids.at[B:2 * B].set(view_ids.astype(jnp.int32))
    rw2p = jnp.pad(rw2, ((0, 0), (0, L - E)))
    rb2p = jnp.pad(rb2, (0, L - E), constant_values=-1e30)
    flat = jnp.concatenate([
        stage_emb.ravel(), view_emb.ravel(), rw1.ravel(), rb1,
        rw2p.ravel(), rb2p])
    mesh = plsc.VectorSubcoreMesh(core_axis_name="c", subcore_axis_name="s")
    fn = functools.partial(
        pl.kernel,
        mesh=mesh,
        out_type=(
            jax.ShapeDtypeStruct((RH // L + B, L), jnp.float32),
            jax.ShapeDtypeStruct((B, L), jnp.int32),
            jax.ShapeDtypeStruct((L,), jnp.float32),
        ),
        scratch_types=[
            pltpu.VMEM((L,), jnp.int32),
            pltpu.VMEM((_FLAT_N,), jnp.float32),
            pltpu.VMEM((RH // L + B, L), jnp.float32),
            pltpu.VMEM((B, L), jnp.int32),
            pltpu.VMEM((L,), jnp.float32),
        ],
    )(_sc_router_body)
    probs16, sel16, loss16 = fn(ids, flat)
    return probs16[:B, :E], sel16[:, 0], loss16[0]


def _ffn_body(sel_ref, x_ref, w1_ref, b1_ref, w2_ref, b2_ref, out_ref):
    j = pl.program_id(1)
    h = jnp.dot(x_ref[0], w1_ref[0], preferred_element_type=jnp.float32)
    h = h + b1_ref[0]
    h = 0.5 * h * (1.0 + jax.lax.erf(h * 0.7071067811865476))
    contrib = jnp.dot(h, w2_ref[0], preferred_element_type=jnp.float32)

    @pl.when(j == 0)
    def _():
        out_ref[0] = contrib + b2_ref[0]

    @pl.when(j > 0)
    def _():
        out_ref[0] += contrib


@jax.jit
def kernel(x, stage_ids, view_ids, stage_emb, view_emb, rw1, rb1, rw2, rb2,
           fc1_w, fc1_b, fc2_w, fc2_b):
    probs, sel, loss = _sc_router(stage_ids, view_ids, stage_emb, view_emb,
                                  rw1, rb1, rw2, rb2)

    grid_spec = pltpu.PrefetchScalarGridSpec(
        num_scalar_prefetch=1,
        grid=(B, NJ),
        in_specs=[
            pl.BlockSpec((1, T, D), lambda b, j, s: (b, 0, 0)),
            pl.BlockSpec((1, D, BF), lambda b, j, s: (s[b], 0, j)),
            pl.BlockSpec((1, 1, BF), lambda b, j, s: (s[b], 0, j)),
            pl.BlockSpec((1, BF, D), lambda b, j, s: (s[b], j, 0)),
            pl.BlockSpec((1, 1, D), lambda b, j, s: (s[b], 0, 0)),
        ],
        out_specs=pl.BlockSpec((1, T, D), lambda b, j, s: (b, 0, 0)),
    )
    output = pl.pallas_call(
        _ffn_body,
        grid_spec=grid_spec,
        out_shape=jax.ShapeDtypeStruct((B, T, D), jnp.float32),
        compiler_params=pltpu.CompilerParams(
            dimension_semantics=("arbitrary", "arbitrary"),
        ),
    )(sel, x, fc1_w, fc1_b.reshape(E, 1, FF), fc2_w, fc2_b.reshape(E, 1, D))

    return output, probs, sel, loss


# speedup vs baseline: 1.0217x; 1.0217x over previous
import functools

import jax
import jax.numpy as jnp
from jax.experimental import pallas as pl
from jax.experimental.pallas import tpu as pltpu
from jax.experimental.pallas import tpu_sc as plsc

B, T, D = 4, 2048, 1024
E = 8
EMB = 64
RH = 128
FF = 4096
NS, NV = 5, 4
L = 16

BF = 512
NJ = FF // BF

_O_SE = 0
_O_VE = _O_SE + NS * EMB
_O_RW1 = _O_VE + NV * EMB
_O_RB1 = _O_RW1 + 2 * EMB * RH
_O_RW2 = _O_RB1 + RH
_O_RB2 = _O_RW2 + RH * L
_FLAT_N = _O_RB2 + L


def _sc_router_body(ids_hbm, flat_hbm, probs_hbm, sel_hbm,
                    ids_v, flat_v, zh_v, row_v, selrow_v):
    cid = jax.lax.axis_index("c")
    scid = jax.lax.axis_index("s")

    @pl.when(jnp.logical_and(cid == 0, scid < B))
    def _():
        b = scid
        pltpu.sync_copy(ids_hbm, ids_v)
        pltpu.sync_copy(flat_hbm, flat_v)
        ids_row = ids_v[...]
        s_id = jnp.int32(0)
        v_id = jnp.int32(0)
        for l in range(B):
            s_id = jnp.where(b == l, ids_row[l], s_id)
            v_id = jnp.where(b == l, ids_row[B + l], v_id)
        for c in range(EMB // L):
            acc = jnp.zeros((L,), jnp.float32)
            for s in range(NS):
                row = flat_v[pl.ds(_O_SE + s * EMB + c * L, L)]
                acc = acc + (s_id == s).astype(jnp.float32) * row
            zh_v[c, :] = acc
        for c in range(EMB // L):
            acc = jnp.zeros((L,), jnp.float32)
            for v in range(NV):
                row = flat_v[pl.ds(_O_VE + v * EMB + c * L, L)]
                acc = acc + (v_id == v).astype(jnp.float32) * row
            zh_v[EMB // L + c, :] = acc

        def l1_body(kc, accs):
            zc = zh_v[kc, :]
            for l in range(L):
                zk = zc[l]
                ko = kc * L + l
                accs = tuple(
                    accs[c] + zk * flat_v[pl.ds(_O_RW1 + ko * RH + c * L, L)]
                    for c in range(RH // L))
            return accs

        h = jax.lax.fori_loop(
            0, 2 * EMB // L, l1_body,
            tuple(flat_v[pl.ds(_O_RB1 + c * L, L)] for c in range(RH // L)))
        for c in range(RH // L):
            zh_v[c, :] = jnp.maximum(h[c], 0.0)

        def l2_body(kc, acc):
            hc = zh_v[kc, :]
            for l in range(L):
                ko = kc * L + l
                acc = acc + hc[l] * flat_v[pl.ds(_O_RW2 + ko * L, L)]
            return acc

        logits = jax.lax.fori_loop(0, RH // L, l2_body,
                                   flat_v[pl.ds(_O_RB2, L)])
        m = logits[0]
        for l in range(1, L):
            m = jnp.maximum(m, logits[l])
        ex = jnp.exp(logits - m)
        ssum = ex[0]
        for l in range(1, L):
            ssum = ssum + ex[l]
        pr = ex / jax.lax.broadcast(ssum, (L,))
        row_v[...] = pr
        selb = jnp.int32(E - 1)
        bestv = pr[E - 1]
        for e in range(E - 2, -1, -1):
            pe = pr[e]
            take = pe >= bestv
            selb = jnp.where(take, jnp.int32(e), selb)
            bestv = jnp.where(take, pe, bestv)
        selrow_v[...] = jax.lax.broadcast(selb, (L,))
        pltpu.sync_copy(row_v, probs_hbm.at[b])
        pltpu.sync_copy(selrow_v, sel_hbm.at[b])


def _sc_router(stage_ids, view_ids, stage_emb, view_emb, rw1, rb1, rw2, rb2):
    ids = jnp.zeros((L,), jnp.int32)
    ids = ids.at[:B].set(stage_ids.astype(jnp.int32))
    ids = ids.at[B:2 * B].set(view_ids.astype(jnp.int32))
    rw2p = jnp.pad(rw2, ((0, 0), (0, L - E)))
    rb2p = jnp.pad(rb2, (0, L - E), constant_values=-1e30)
    flat = jnp.concatenate([
        stage_emb.ravel(), view_emb.ravel(), rw1.ravel(), rb1,
        rw2p.ravel(), rb2p])
    mesh = plsc.VectorSubcoreMesh(core_axis_name="c", subcore_axis_name="s")
    fn = functools.partial(
        pl.kernel,
        mesh=mesh,
        out_type=(
            jax.ShapeDtypeStruct((B, L), jnp.float32),
            jax.ShapeDtypeStruct((B, L), jnp.int32),
        ),
        scratch_types=[
            pltpu.VMEM((L,), jnp.int32),
            pltpu.VMEM((_FLAT_N,), jnp.float32),
            pltpu.VMEM((RH // L, L), jnp.float32),
            pltpu.VMEM((L,), jnp.float32),
            pltpu.VMEM((L,), jnp.int32),
        ],
    )(_sc_router_body)
    probs16, sel16 = fn(ids, flat)
    return probs16, sel16[:, 0]


def _ffn_body(sel_ref, x_ref, w1_ref, b1_ref, w2_ref, b2_ref, probs_ref,
              out_ref, loss_ref):
    j = pl.program_id(1)
    bi = pl.program_id(0)

    @pl.when(jnp.logical_and(bi == 0, j == 0))
    def _():
        rows_i = jax.lax.broadcasted_iota(jnp.int32, (B, E), 0)
        cols_i = jax.lax.broadcasted_iota(jnp.int32, (B, E), 1)
        selmat = jnp.zeros((B, E), jnp.int32)
        for b in range(B):
            selmat = jnp.where(rows_i == b, sel_ref[b], selmat)
        oh = (cols_i == selmat).astype(jnp.float32)
        f = jnp.mean(oh, axis=0)
        P = jnp.mean(probs_ref[:, :E], axis=0)
        loss_ref[...] = (E * jnp.sum(f * P)).reshape(1, 1)

    h = jnp.dot(x_ref[0], w1_ref[0], preferred_element_type=jnp.float32)
    h = h + b1_ref[0]
    h = 0.5 * h * (1.0 + jax.lax.erf(h * 0.7071067811865476))
    contrib = jnp.dot(h, w2_ref[0], preferred_element_type=jnp.float32)

    @pl.when(j == 0)
    def _():
        out_ref[0] = contrib + b2_ref[0]

    @pl.when(j > 0)
    def _():
        out_ref[0] += contrib


@jax.jit
def kernel(x, stage_ids, view_ids, stage_emb, view_emb, rw1, rb1, rw2, rb2,
           fc1_w, fc1_b, fc2_w, fc2_b):
    probs16, sel = _sc_router(stage_ids, view_ids, stage_emb, view_emb,
                              rw1, rb1, rw2, rb2)

    grid_spec = pltpu.PrefetchScalarGridSpec(
        num_scalar_prefetch=1,
        grid=(B, NJ),
        in_specs=[
            pl.BlockSpec((1, T, D), lambda b, j, s: (b, 0, 0)),
            pl.BlockSpec((1, D, BF), lambda b, j, s: (s[b], 0, j)),
            pl.BlockSpec((1, 1, BF), lambda b, j, s: (s[b], 0, j)),
            pl.BlockSpec((1, BF, D), lambda b, j, s: (s[b], j, 0)),
            pl.BlockSpec((1, 1, D), lambda b, j, s: (s[b], 0, 0)),
            pl.BlockSpec((B, L), lambda b, j, s: (0, 0)),
        ],
        out_specs=(
            pl.BlockSpec((1, T, D), lambda b, j, s: (b, 0, 0)),
            pl.BlockSpec((1, 1), lambda b, j, s: (0, 0)),
        ),
    )
    output, loss2d = pl.pallas_call(
        _ffn_body,
        grid_spec=grid_spec,
        out_shape=(
            jax.ShapeDtypeStruct((B, T, D), jnp.float32),
            jax.ShapeDtypeStruct((1, 1), jnp.float32),
        ),
        compiler_params=pltpu.CompilerParams(
            dimension_semantics=("arbitrary", "arbitrary"),
        ),
    )(sel, x, fc1_w, fc1_b.reshape(E, 1, FF), fc2_w, fc2_b.reshape(E, 1, D),
      probs16)

    return output, probs16[:, :E], sel, loss2d[0, 0]
